# baseline (speedup 1.0000x reference)
import jax
import jax.numpy as jnp
from jax.experimental import pallas as pl
from jax.experimental.pallas import tpu as pltpu

_BR = 400
_NBIG = 24
_MR = 80
_NMINI = 5


def _body(h_hbm, w_ref, adj_hbm, out_hbm,
          h_buf, hw_scr, adj_buf, mini_buf, out_buf, mout_buf,
          h_sem, adj_sem, mini_sem, out_sem, mout_sem):

    def adj_copy(i, slot):
        return pltpu.make_async_copy(
            adj_hbm.at[pl.ds(i * _BR, _BR), :], adj_buf.at[slot],
            adj_sem.at[slot])

    def mini_copy(j, slot):
        return pltpu.make_async_copy(
            adj_hbm.at[pl.ds(_NBIG * _BR + j * _MR, _MR), :],
            mini_buf.at[slot], mini_sem.at[slot])

    def out_copy(i, slot):
        return pltpu.make_async_copy(
            out_buf.at[slot], out_hbm.at[pl.ds(i * _BR, _BR), :],
            out_sem.at[slot])

    def mout_copy(j, slot):
        return pltpu.make_async_copy(
            mout_buf.at[slot], out_hbm.at[pl.ds(_NBIG * _BR + j * _MR, _MR), :],
            mout_sem.at[slot])

    h_cp = pltpu.make_async_copy(h_hbm, h_buf, h_sem)
    h_cp.start()
    adj_copy(0, 0).start()
    adj_copy(1, 1).start()
    h_cp.wait()
    hw_scr[...] = jnp.dot(
        h_buf[...].astype(jnp.bfloat16),
        w_ref[...].astype(jnp.bfloat16),
        preferred_element_type=jnp.float32,
    ).astype(jnp.bfloat16)

    for i in range(_NBIG):
        slot = i % 2
        adj_copy(i, slot).wait()
        if i >= 2:
            out_copy(i - 2, slot).wait()
        out_buf[slot] = jnp.dot(
            adj_buf[slot].astype(jnp.bfloat16), hw_scr[...],
            preferred_element_type=jnp.float32)
        out_copy(i, slot).start()
        if i + 2 < _NBIG:
            adj_copy(i + 2, slot).start()
        elif i + 2 == _NBIG:
            mini_copy(0, 0).start()
        elif i + 2 == _NBIG + 1:
            mini_copy(1, 1).start()

    for j in range(_NMINI):
        slot = j % 2
        mini_copy(j, slot).wait()
        if j >= 2:
            mout_copy(j - 2, slot).wait()
        mout_buf[slot] = jnp.dot(
            mini_buf[slot].astype(jnp.bfloat16), hw_scr[...],
            preferred_element_type=jnp.float32)
        mout_copy(j, slot).start()
        if j + 2 < _NMINI:
            mini_copy(j + 2, slot).start()

    out_copy(_NBIG - 2, (_NBIG - 2) % 2).wait()
    out_copy(_NBIG - 1, (_NBIG - 1) % 2).wait()
    mout_copy(_NMINI - 2, (_NMINI - 2) % 2).wait()
    mout_copy(_NMINI - 1, (_NMINI - 1) % 2).wait()


def kernel(h, adj, W):
    n, in_f = h.shape
    out_f = W.shape[1]

    out = pl.pallas_call(
        _body,
        in_specs=[
            pl.BlockSpec(memory_space=pl.ANY),
            pl.BlockSpec(memory_space=pltpu.MemorySpace.VMEM),
            pl.BlockSpec(memory_space=pl.ANY),
        ],
        out_specs=pl.BlockSpec(memory_space=pl.ANY),
        out_shape=jax.ShapeDtypeStruct((n, out_f), jnp.float32),
        scratch_shapes=[
            pltpu.VMEM((n, in_f), jnp.float32),
            pltpu.VMEM((n, out_f), jnp.bfloat16),
            pltpu.VMEM((2, _BR, n), jnp.float32),
            pltpu.VMEM((2, _MR, n), jnp.float32),
            pltpu.VMEM((2, _BR, out_f), jnp.float32),
            pltpu.VMEM((2, _MR, out_f), jnp.float32),
            pltpu.SemaphoreType.DMA,
            pltpu.SemaphoreType.DMA((2,)),
            pltpu.SemaphoreType.DMA((2,)),
            pltpu.SemaphoreType.DMA((2,)),
            pltpu.SemaphoreType.DMA((2,)),
        ],
    )(h, W, adj)
    return out


# speedup vs baseline: 1.1026x; 1.1026x over previous
import jax
import jax.numpy as jnp
from jax.experimental import pallas as pl
from jax.experimental.pallas import tpu as pltpu


def _body(h_ref, w_ref, adj_ref, out_ref, hw_scr):
    i = pl.program_id(0)

    @pl.when(i == 0)
    def _hw():
        hw_scr[...] = jnp.dot(
            h_ref[...].astype(jnp.bfloat16),
            w_ref[...].astype(jnp.bfloat16),
            preferred_element_type=jnp.float32,
        ).astype(jnp.bfloat16)

    out_ref[...] = jnp.dot(
        adj_ref[...].astype(jnp.bfloat16),
        hw_scr[...],
        preferred_element_type=jnp.float32,
    )


def kernel(h, adj, W):
    n, in_f = h.shape
    out_f = W.shape[1]
    br = 400

    out = pl.pallas_call(
        _body,
        grid=(n // br,),
        in_specs=[
            pl.BlockSpec((n, in_f), lambda i: (0, 0)),
            pl.BlockSpec((in_f, out_f), lambda i: (0, 0)),
            pl.BlockSpec((br, n), lambda i: (i, 0)),
        ],
        out_specs=pl.BlockSpec((br, out_f), lambda i: (i, 0)),
        out_shape=jax.ShapeDtypeStruct((n, out_f), jnp.float32),
        scratch_shapes=[pltpu.VMEM((n, out_f), jnp.bfloat16)],
        compiler_params=pltpu.CompilerParams(
            dimension_semantics=("arbitrary",),
        ),
    )(h, W, adj)
    return out
